# Initial kernel scaffold; baseline (speedup 1.0000x reference)
import jax
import jax.numpy as jnp
from jax.experimental import pallas as pl


def kernel(in_feat, edge_index, W1, b1, W2, b2):
    raise NotImplementedError("write your pallas kernel here")


# speedup vs baseline: 13.6427x; 13.6427x over previous
import functools

import jax
import jax.numpy as jnp
from jax import lax
from jax.experimental import pallas as pl
from jax.experimental.pallas import tpu as pltpu
from jax.experimental.pallas import tpu_sc as plsc

N = 10000
NPAD = 10240
ROWS_PT = NPAD // 16
EPB = 128
NB = 80
NTILES = 32
EPAD = NTILES * NB * EPB
DEGW = 128

_mesh = plsc.VectorSubcoreMesh(core_axis_name="c", subcore_axis_name="s")


@functools.partial(
    pl.kernel,
    out_type=jax.ShapeDtypeStruct((2, NPAD, DEGW), jnp.float32),
    mesh=_mesh,
    scratch_types=[
        pltpu.VMEM((NB, EPB), jnp.int32),
        pltpu.VMEM((EPB, DEGW), jnp.float32),
        pltpu.VMEM_SHARED((NPAD, DEGW), jnp.float32),
    ],
)
def _deg_kernel(dst_hbm, ones_hbm, zeros_hbm, out_hbm, dst_v, ones_v, acc_sh):
    c = lax.axis_index("c")
    s = lax.axis_index("s")
    w = c * 16 + s
    pltpu.sync_copy(dst_hbm.at[w], dst_v)
    pltpu.sync_copy(ones_hbm, ones_v)
    pltpu.sync_copy(zeros_hbm.at[pl.ds(s * ROWS_PT, ROWS_PT)],
                    acc_sh.at[pl.ds(s * ROWS_PT, ROWS_PT)])
    plsc.subcore_barrier()

    @pl.loop(0, NB)
    def _(j):
        pltpu.sync_copy(ones_v, acc_sh.at[dst_v.at[j]], add=True)

    plsc.subcore_barrier()
    pltpu.sync_copy(acc_sh.at[pl.ds(s * ROWS_PT, ROWS_PT)],
                    out_hbm.at[c, pl.ds(s * ROWS_PT, ROWS_PT)])


SCH = 16
NSC = NB // SCH


def _make_agg_kernel(D):
    @functools.partial(
        pl.kernel,
        out_type=jax.ShapeDtypeStruct((2, NPAD, D), jnp.float32),
        mesh=_mesh,
        scratch_types=[
            pltpu.VMEM((SCH, EPB), jnp.int32),
            pltpu.VMEM((SCH, EPB), jnp.int32),
            pltpu.VMEM((EPB, D), jnp.float32),
            pltpu.VMEM((EPB, D), jnp.float32),
            pltpu.VMEM_SHARED((NPAD, D), jnp.float32),
            pltpu.SemaphoreType.DMA,
            pltpu.SemaphoreType.DMA,
        ],
    )
    def agg(y_hbm, src_hbm, dst_hbm, zeros_hbm, out_hbm,
            src_v, dst_v, buf0, buf1, acc_sh, sem0, sem1):
        c = lax.axis_index("c")
        s = lax.axis_index("s")
        w = c * 16 + s
        pltpu.sync_copy(zeros_hbm.at[pl.ds(s * ROWS_PT, ROWS_PT)],
                        acc_sh.at[pl.ds(s * ROWS_PT, ROWS_PT)])
        plsc.subcore_barrier()

        bufs = (buf0, buf1)
        sems = (sem0, sem1)

        @pl.loop(0, NSC)
        def _(sc):
            pltpu.sync_copy(src_hbm.at[w, pl.ds(sc * SCH, SCH)], src_v)
            pltpu.sync_copy(dst_hbm.at[w, pl.ds(sc * SCH, SCH)], dst_v)
            pltpu.async_copy(y_hbm.at[src_v.at[0]], buf0, sem0)

            @pl.loop(0, SCH // 2)
            def _(g):
                for b in range(2):
                    j = g * 2 + b

                    @pl.when(j + 1 < SCH)
                    def _():
                        pltpu.async_copy(y_hbm.at[src_v.at[j + 1]],
                                         bufs[1 - b], sems[1 - b])

                    pltpu.make_async_copy(y_hbm.at[src_v.at[j]], bufs[b],
                                          sems[b]).wait()
                    pltpu.sync_copy(bufs[b], acc_sh.at[dst_v.at[j]], add=True)

        plsc.subcore_barrier()
        pltpu.sync_copy(acc_sh.at[pl.ds(s * ROWS_PT, ROWS_PT)],
                        out_hbm.at[c, pl.ds(s * ROWS_PT, ROWS_PT)])

    return agg


_agg128_kernel = _make_agg_kernel(128)


def _t0_body(degp_ref, x_ref, dinv_ref, y_ref):
    deg8 = degp_ref[0] + degp_ref[1] + 1.0
    dinv8 = lax.rsqrt(deg8)
    d = dinv8[:, 0:1]
    dinv_ref[...] = d
    y_ref[...] = x_ref[...] * d


def _t1_body(sp_ref, x_ref, dinv_ref, w1_ref, b1_ref, w2_ref, z_ref, y2_ref):
    d = dinv_ref[...]
    agg = d * (sp_ref[0] + sp_ref[1]) + (d * d) * x_ref[...]
    h = jnp.dot(agg, w1_ref[...], preferred_element_type=jnp.float32) + b1_ref[...]
    h = jnp.where(h >= 0, h, 0.01 * h)
    z = jnp.dot(h, w2_ref[...], preferred_element_type=jnp.float32)
    z_ref[...] = z
    y2_ref[...] = z * d


def _t2_body(sp_ref, z_ref, dinv_ref, b2_ref, out_ref):
    d = dinv_ref[...]
    o = (d * (sp_ref[0][:, 0:4] + sp_ref[1][:, 0:4])
         + (d * d) * z_ref[:, 0:4] + b2_ref[...])
    m = jnp.max(o, axis=1, keepdims=True)
    e = jnp.exp(o - m)
    out_ref[...] = o - m - jnp.log(jnp.sum(e, axis=1, keepdims=True))


def kernel(in_feat, edge_index, W1, b1, W2, b2):
    src = edge_index[0].astype(jnp.int32)
    dst = edge_index[1].astype(jnp.int32)
    e = src.shape[0]
    pad = jnp.full((EPAD - e,), N, jnp.int32)
    src_p = jnp.concatenate([src, pad]).reshape(NTILES, NB, EPB)
    dst_p = jnp.concatenate([dst, pad]).reshape(NTILES, NB, EPB)

    ones1 = jnp.zeros((EPB, DEGW), jnp.float32).at[:, 0].set(1.0)
    zeros128 = jnp.zeros((NPAD, 128), jnp.float32)
    x_pad = jnp.zeros((NPAD, 128), jnp.float32).at[:N].set(in_feat)
    w2p = jnp.zeros((512, 128), jnp.float32).at[:, :4].set(W2)

    degp = _deg_kernel(dst_p, ones1, zeros128)

    dinv, y1 = pl.pallas_call(
        _t0_body,
        out_shape=[jax.ShapeDtypeStruct((NPAD, 1), jnp.float32),
                   jax.ShapeDtypeStruct((NPAD, 128), jnp.float32)],
    )(degp, x_pad)

    s1p = _agg128_kernel(y1, src_p, dst_p, zeros128)

    R = 1280
    z, y2 = pl.pallas_call(
        _t1_body,
        grid=(NPAD // R,),
        in_specs=[
            pl.BlockSpec((2, R, 128), lambda i: (0, i, 0)),
            pl.BlockSpec((R, 128), lambda i: (i, 0)),
            pl.BlockSpec((R, 1), lambda i: (i, 0)),
            pl.BlockSpec((128, 512), lambda i: (0, 0)),
            pl.BlockSpec((1, 512), lambda i: (0, 0)),
            pl.BlockSpec((512, 128), lambda i: (0, 0)),
        ],
        out_specs=[
            pl.BlockSpec((R, 128), lambda i: (i, 0)),
            pl.BlockSpec((R, 128), lambda i: (i, 0)),
        ],
        out_shape=[jax.ShapeDtypeStruct((NPAD, 128), jnp.float32),
                   jax.ShapeDtypeStruct((NPAD, 128), jnp.float32)],
    )(s1p, x_pad, dinv, W1, b1.reshape(1, 512), w2p)

    s2p = _agg128_kernel(y2, src_p, dst_p, zeros128)

    out = pl.pallas_call(
        _t2_body,
        out_shape=jax.ShapeDtypeStruct((NPAD, 4), jnp.float32),
    )(s2p, z, dinv, b2.reshape(1, 4))
    return out[:N]


# speedup vs baseline: 33.2495x; 2.4372x over previous
import functools

import jax
import jax.numpy as jnp
from jax import lax
from jax.experimental import pallas as pl
from jax.experimental.pallas import tpu as pltpu
from jax.experimental.pallas import tpu_sc as plsc

N = 10000
NPAD = 10240
ROWS_PT = NPAD // 16
EPB = 128
NB = 80
NTILES = 32
EPAD = NTILES * NB * EPB
DEGW = 128

_mesh = plsc.VectorSubcoreMesh(core_axis_name="c", subcore_axis_name="s")


@functools.partial(
    pl.kernel,
    out_type=jax.ShapeDtypeStruct((2, NPAD, DEGW), jnp.float32),
    mesh=_mesh,
    scratch_types=[
        pltpu.VMEM((NB, EPB), jnp.int32),
        pltpu.VMEM((EPB, DEGW), jnp.float32),
        pltpu.VMEM_SHARED((NPAD, DEGW), jnp.float32),
    ],
)
def _deg_kernel(dst_hbm, ones_hbm, zeros_hbm, out_hbm, dst_v, ones_v, acc_sh):
    c = lax.axis_index("c")
    s = lax.axis_index("s")
    w = c * 16 + s
    pltpu.sync_copy(dst_hbm.at[w], dst_v)
    pltpu.sync_copy(ones_hbm, ones_v)
    pltpu.sync_copy(zeros_hbm.at[pl.ds(s * ROWS_PT, ROWS_PT)],
                    acc_sh.at[pl.ds(s * ROWS_PT, ROWS_PT)])
    plsc.subcore_barrier()

    @pl.loop(0, NB)
    def _(j):
        pltpu.sync_copy(ones_v, acc_sh.at[dst_v.at[j]], add=True)

    plsc.subcore_barrier()
    pltpu.sync_copy(acc_sh.at[pl.ds(s * ROWS_PT, ROWS_PT)],
                    out_hbm.at[c, pl.ds(s * ROWS_PT, ROWS_PT)])


SCH = 16
NSC = NB // SCH


def _make_agg_kernel(D):
    @functools.partial(
        pl.kernel,
        out_type=jax.ShapeDtypeStruct((2, NPAD, D), jnp.float32),
        mesh=_mesh,
        scratch_types=[
            pltpu.VMEM((SCH, EPB), jnp.int32),
            pltpu.VMEM((SCH, EPB), jnp.int32),
            pltpu.VMEM((EPB, D), jnp.float32),
            pltpu.VMEM((EPB, D), jnp.float32),
            pltpu.VMEM_SHARED((NPAD, D), jnp.float32),
            pltpu.SemaphoreType.DMA,
            pltpu.SemaphoreType.DMA,
        ],
    )
    def agg(y_hbm, src_hbm, dst_hbm, zeros_hbm, out_hbm,
            src_v, dst_v, buf0, buf1, acc_sh, sem0, sem1):
        c = lax.axis_index("c")
        s = lax.axis_index("s")
        w = c * 16 + s
        pltpu.sync_copy(zeros_hbm.at[pl.ds(s * ROWS_PT, ROWS_PT)],
                        acc_sh.at[pl.ds(s * ROWS_PT, ROWS_PT)])
        plsc.subcore_barrier()

        bufs = (buf0, buf1)
        sems = (sem0, sem1)

        @pl.loop(0, NSC)
        def _(sc):
            pltpu.sync_copy(src_hbm.at[w, pl.ds(sc * SCH, SCH)], src_v)
            pltpu.sync_copy(dst_hbm.at[w, pl.ds(sc * SCH, SCH)], dst_v)
            pltpu.async_copy(y_hbm.at[src_v.at[0]], buf0, sem0)

            @pl.loop(0, SCH // 2)
            def _(g):
                for b in range(2):
                    j = g * 2 + b

                    @pl.when(j + 1 < SCH)
                    def _():
                        pltpu.async_copy(y_hbm.at[src_v.at[j + 1]],
                                         bufs[1 - b], sems[1 - b])

                    pltpu.make_async_copy(y_hbm.at[src_v.at[j]], bufs[b],
                                          sems[b]).wait()
                    pltpu.sync_copy(bufs[b], acc_sh.at[dst_v.at[j]], add=True)

        plsc.subcore_barrier()
        pltpu.sync_copy(acc_sh.at[pl.ds(s * ROWS_PT, ROWS_PT)],
                        out_hbm.at[c, pl.ds(s * ROWS_PT, ROWS_PT)])

    return agg


_agg128_kernel = _make_agg_kernel(128)


def _t0_body(degp_ref, x_ref, dinv_ref, y_ref):
    deg8 = degp_ref[0] + degp_ref[1] + 1.0
    dinv8 = lax.rsqrt(deg8)
    d = dinv8[:, 0:1]
    dinv_ref[...] = d
    y_ref[...] = x_ref[...] * d


def _t1_body(sp_ref, x_ref, dinv_ref, w1_ref, b1_ref, w2_ref, z_ref, y2_ref):
    d = dinv_ref[...]
    agg = d * (sp_ref[0] + sp_ref[1]) + (d * d) * x_ref[...]
    h = jnp.dot(agg, w1_ref[...], preferred_element_type=jnp.float32) + b1_ref[...]
    h = jnp.where(h >= 0, h, 0.01 * h)
    z = jnp.dot(h, w2_ref[...], preferred_element_type=jnp.float32)
    z_ref[...] = z
    y2_ref[...] = z * d


def _t2_body(sp_ref, z_ref, dinv_ref, b2_ref, out_ref):
    d = dinv_ref[...]
    o = (d * (sp_ref[0][:, 0:4] + sp_ref[1][:, 0:4])
         + (d * d) * z_ref[:, 0:4] + b2_ref[...])
    m = jnp.max(o, axis=1, keepdims=True)
    e = jnp.exp(o - m)
    out_ref[...] = o - m - jnp.log(jnp.sum(e, axis=1, keepdims=True))


def kernel(in_feat, edge_index, W1, b1, W2, b2):
    src = edge_index[0].astype(jnp.int32)
    dst = edge_index[1].astype(jnp.int32)
    e = src.shape[0]
    pad_i = jnp.arange(EPAD - e, dtype=jnp.int32)
    pad_src = N + pad_i % (NPAD - N)
    pad_dst = pad_i % NPAD
    src_p = jnp.concatenate([src, pad_src]).reshape(NTILES, NB, EPB)
    dst_p = jnp.concatenate([dst, pad_dst]).reshape(NTILES, NB, EPB)

    ones1 = jnp.zeros((EPB, DEGW), jnp.float32).at[:, 0].set(1.0)
    zeros128 = jnp.zeros((NPAD, 128), jnp.float32)
    x_pad = jnp.zeros((NPAD, 128), jnp.float32).at[:N].set(in_feat)
    w2p = jnp.zeros((512, 128), jnp.float32).at[:, :4].set(W2)

    degp = _deg_kernel(dst_p, ones1, zeros128)

    dinv, y1 = pl.pallas_call(
        _t0_body,
        out_shape=[jax.ShapeDtypeStruct((NPAD, 1), jnp.float32),
                   jax.ShapeDtypeStruct((NPAD, 128), jnp.float32)],
    )(degp, x_pad)

    s1p = _agg128_kernel(y1, src_p, dst_p, zeros128)

    R = 1280
    z, y2 = pl.pallas_call(
        _t1_body,
        grid=(NPAD // R,),
        in_specs=[
            pl.BlockSpec((2, R, 128), lambda i: (0, i, 0)),
            pl.BlockSpec((R, 128), lambda i: (i, 0)),
            pl.BlockSpec((R, 1), lambda i: (i, 0)),
            pl.BlockSpec((128, 512), lambda i: (0, 0)),
            pl.BlockSpec((1, 512), lambda i: (0, 0)),
            pl.BlockSpec((512, 128), lambda i: (0, 0)),
        ],
        out_specs=[
            pl.BlockSpec((R, 128), lambda i: (i, 0)),
            pl.BlockSpec((R, 128), lambda i: (i, 0)),
        ],
        out_shape=[jax.ShapeDtypeStruct((NPAD, 128), jnp.float32),
                   jax.ShapeDtypeStruct((NPAD, 128), jnp.float32)],
    )(s1p, x_pad, dinv, W1, b1.reshape(1, 512), w2p)

    s2p = _agg128_kernel(y2, src_p, dst_p, zeros128)

    out = pl.pallas_call(
        _t2_body,
        out_shape=jax.ShapeDtypeStruct((NPAD, 4), jnp.float32),
    )(s2p, z, dinv, b2.reshape(1, 4))
    return out[:N]


# speedup vs baseline: 33.2941x; 1.0013x over previous
import functools

import jax
import jax.numpy as jnp
from jax import lax
from jax.experimental import pallas as pl
from jax.experimental.pallas import tpu as pltpu
from jax.experimental.pallas import tpu_sc as plsc

N = 10000
NPAD = 10240
ROWS_PT = NPAD // 16
EPB = 128
NB = 80
NTILES = 32
EPAD = NTILES * NB * EPB
DEGW = 128

_mesh = plsc.VectorSubcoreMesh(core_axis_name="c", subcore_axis_name="s")


@functools.partial(
    pl.kernel,
    out_type=jax.ShapeDtypeStruct((2, NPAD, DEGW), jnp.float32),
    mesh=_mesh,
    scratch_types=[
        pltpu.VMEM((NB, EPB), jnp.int32),
        pltpu.VMEM((EPB, DEGW), jnp.float32),
        pltpu.VMEM_SHARED((NPAD, DEGW), jnp.float32),
    ],
)
def _deg_kernel(dst_hbm, ones_hbm, zeros_hbm, out_hbm, dst_v, ones_v, acc_sh):
    c = lax.axis_index("c")
    s = lax.axis_index("s")
    w = c * 16 + s
    pltpu.sync_copy(dst_hbm.at[w], dst_v)
    pltpu.sync_copy(ones_hbm, ones_v)
    pltpu.sync_copy(zeros_hbm.at[pl.ds(s * ROWS_PT, ROWS_PT)],
                    acc_sh.at[pl.ds(s * ROWS_PT, ROWS_PT)])
    plsc.subcore_barrier()

    @pl.loop(0, NB)
    def _(j):
        pltpu.sync_copy(ones_v, acc_sh.at[dst_v.at[j]], add=True)

    plsc.subcore_barrier()
    pltpu.sync_copy(acc_sh.at[pl.ds(s * ROWS_PT, ROWS_PT)],
                    out_hbm.at[c, pl.ds(s * ROWS_PT, ROWS_PT)])


SCH = 16
NSC = NB // SCH


def _make_agg_kernel(D):
    @functools.partial(
        pl.kernel,
        out_type=jax.ShapeDtypeStruct((2, NPAD, D), jnp.float32),
        mesh=_mesh,
        scratch_types=[
            pltpu.VMEM((SCH, EPB), jnp.int32),
            pltpu.VMEM((SCH, EPB), jnp.int32),
            pltpu.VMEM((EPB, D), jnp.float32),
            pltpu.VMEM((EPB, D), jnp.float32),
            pltpu.VMEM_SHARED((NPAD, D), jnp.float32),
            pltpu.SemaphoreType.DMA,
            pltpu.SemaphoreType.DMA,
            pltpu.SemaphoreType.DMA,
            pltpu.SemaphoreType.DMA,
        ],
    )
    def agg(y_hbm, src_hbm, dst_hbm, zeros_hbm, out_hbm,
            src_v, dst_v, buf0, buf1, acc_sh, semg0, semg1, sems0, sems1):
        c = lax.axis_index("c")
        s = lax.axis_index("s")
        w = c * 16 + s
        pltpu.sync_copy(zeros_hbm.at[pl.ds(s * ROWS_PT, ROWS_PT)],
                        acc_sh.at[pl.ds(s * ROWS_PT, ROWS_PT)])
        plsc.subcore_barrier()

        bufs = (buf0, buf1)
        semg = (semg0, semg1)
        sems = (sems0, sems1)

        @pl.loop(0, NSC)
        def _(sc):
            pltpu.sync_copy(src_hbm.at[w, pl.ds(sc * SCH, SCH)], src_v)
            pltpu.sync_copy(dst_hbm.at[w, pl.ds(sc * SCH, SCH)], dst_v)
            pltpu.async_copy(y_hbm.at[src_v.at[0]], buf0, semg0)

            @pl.loop(0, SCH // 2)
            def _(g):
                for b in range(2):
                    j = g * 2 + b

                    @pl.when(j >= 1)
                    def _():
                        pltpu.make_async_copy(
                            bufs[1 - b], acc_sh.at[dst_v.at[j - 1]],
                            sems[1 - b]).wait()

                    @pl.when(j + 1 < SCH)
                    def _():
                        pltpu.async_copy(y_hbm.at[src_v.at[j + 1]],
                                         bufs[1 - b], semg[1 - b])

                    pltpu.make_async_copy(y_hbm.at[src_v.at[j]], bufs[b],
                                          semg[b]).wait()
                    pltpu.async_copy(bufs[b], acc_sh.at[dst_v.at[j]],
                                     sems[b], add=True)

            pltpu.make_async_copy(bufs[1], acc_sh.at[dst_v.at[SCH - 1]],
                                  sems[1]).wait()

        plsc.subcore_barrier()
        pltpu.sync_copy(acc_sh.at[pl.ds(s * ROWS_PT, ROWS_PT)],
                        out_hbm.at[c, pl.ds(s * ROWS_PT, ROWS_PT)])

    return agg


_agg128_kernel = _make_agg_kernel(128)


def _t0_body(degp_ref, x_ref, dinv_ref, y_ref):
    deg8 = degp_ref[0] + degp_ref[1] + 1.0
    dinv8 = lax.rsqrt(deg8)
    d = dinv8[:, 0:1]
    dinv_ref[...] = d
    y_ref[...] = x_ref[...] * d


def _t1_body(sp_ref, x_ref, dinv_ref, w1_ref, b1_ref, w2_ref, z_ref, y2_ref):
    d = dinv_ref[...]
    agg = d * (sp_ref[0] + sp_ref[1]) + (d * d) * x_ref[...]
    h = jnp.dot(agg, w1_ref[...], preferred_element_type=jnp.float32) + b1_ref[...]
    h = jnp.where(h >= 0, h, 0.01 * h)
    z = jnp.dot(h, w2_ref[...], preferred_element_type=jnp.float32)
    z_ref[...] = z
    y2_ref[...] = z * d


def _t2_body(sp_ref, z_ref, dinv_ref, b2_ref, out_ref):
    d = dinv_ref[...]
    o = (d * (sp_ref[0][:, 0:4] + sp_ref[1][:, 0:4])
         + (d * d) * z_ref[:, 0:4] + b2_ref[...])
    m = jnp.max(o, axis=1, keepdims=True)
    e = jnp.exp(o - m)
    out_ref[...] = o - m - jnp.log(jnp.sum(e, axis=1, keepdims=True))


def kernel(in_feat, edge_index, W1, b1, W2, b2):
    src = edge_index[0].astype(jnp.int32)
    dst = edge_index[1].astype(jnp.int32)
    e = src.shape[0]
    pad_i = jnp.arange(EPAD - e, dtype=jnp.int32)
    pad_src = N + pad_i % (NPAD - N)
    pad_dst = pad_i % NPAD
    src_p = jnp.concatenate([src, pad_src]).reshape(NTILES, NB, EPB)
    dst_p = jnp.concatenate([dst, pad_dst]).reshape(NTILES, NB, EPB)

    ones1 = jnp.zeros((EPB, DEGW), jnp.float32).at[:, 0].set(1.0)
    zeros128 = jnp.zeros((NPAD, 128), jnp.float32)
    x_pad = jnp.zeros((NPAD, 128), jnp.float32).at[:N].set(in_feat)
    w2p = jnp.zeros((512, 128), jnp.float32).at[:, :4].set(W2)

    degp = _deg_kernel(dst_p, ones1, zeros128)

    dinv, y1 = pl.pallas_call(
        _t0_body,
        out_shape=[jax.ShapeDtypeStruct((NPAD, 1), jnp.float32),
                   jax.ShapeDtypeStruct((NPAD, 128), jnp.float32)],
    )(degp, x_pad)

    s1p = _agg128_kernel(y1, src_p, dst_p, zeros128)

    R = 1280
    z, y2 = pl.pallas_call(
        _t1_body,
        grid=(NPAD // R,),
        in_specs=[
            pl.BlockSpec((2, R, 128), lambda i: (0, i, 0)),
            pl.BlockSpec((R, 128), lambda i: (i, 0)),
            pl.BlockSpec((R, 1), lambda i: (i, 0)),
            pl.BlockSpec((128, 512), lambda i: (0, 0)),
            pl.BlockSpec((1, 512), lambda i: (0, 0)),
            pl.BlockSpec((512, 128), lambda i: (0, 0)),
        ],
        out_specs=[
            pl.BlockSpec((R, 128), lambda i: (i, 0)),
            pl.BlockSpec((R, 128), lambda i: (i, 0)),
        ],
        out_shape=[jax.ShapeDtypeStruct((NPAD, 128), jnp.float32),
                   jax.ShapeDtypeStruct((NPAD, 128), jnp.float32)],
    )(s1p, x_pad, dinv, W1, b1.reshape(1, 512), w2p)

    s2p = _agg128_kernel(y2, src_p, dst_p, zeros128)

    out = pl.pallas_call(
        _t2_body,
        out_shape=jax.ShapeDtypeStruct((NPAD, 4), jnp.float32),
    )(s2p, z, dinv, b2.reshape(1, 4))
    return out[:N]


# speedup vs baseline: 33.5207x; 1.0068x over previous
import functools

import jax
import jax.numpy as jnp
from jax import lax
from jax.experimental import pallas as pl
from jax.experimental.pallas import tpu as pltpu
from jax.experimental.pallas import tpu_sc as plsc

N = 10000
NPAD = 10240
ROWS_PT = NPAD // 16
EPB = 128
NB = 80
NTILES = 32
EPAD = NTILES * NB * EPB
DEGW = 128

_mesh = plsc.VectorSubcoreMesh(core_axis_name="c", subcore_axis_name="s")


@functools.partial(
    pl.kernel,
    out_type=jax.ShapeDtypeStruct((2, NPAD, DEGW), jnp.float32),
    mesh=_mesh,
    scratch_types=[
        pltpu.VMEM((NB, EPB), jnp.int32),
        pltpu.VMEM((EPB, DEGW), jnp.float32),
        pltpu.VMEM_SHARED((NPAD, DEGW), jnp.float32),
        pltpu.SemaphoreType.DMA,
        pltpu.SemaphoreType.DMA,
    ],
)
def _deg_kernel(dst_hbm, ones_hbm, zeros_hbm, out_hbm, dst_v, ones_v, acc_sh,
                sem0, sem1):
    c = lax.axis_index("c")
    s = lax.axis_index("s")
    w = c * 16 + s
    pltpu.sync_copy(dst_hbm.at[w], dst_v)
    pltpu.sync_copy(ones_hbm, ones_v)
    pltpu.sync_copy(zeros_hbm.at[pl.ds(s * ROWS_PT, ROWS_PT)],
                    acc_sh.at[pl.ds(s * ROWS_PT, ROWS_PT)])
    plsc.subcore_barrier()

    sems = (sem0, sem1)

    @pl.loop(0, NB // 2)
    def _(g):
        for b in range(2):
            j = g * 2 + b

            @pl.when(j >= 2)
            def _():
                pltpu.make_async_copy(ones_v, acc_sh.at[dst_v.at[j - 2]],
                                      sems[b]).wait()

            pltpu.async_copy(ones_v, acc_sh.at[dst_v.at[j]], sems[b], add=True)

    pltpu.make_async_copy(ones_v, acc_sh.at[dst_v.at[NB - 2]], sems[0]).wait()
    pltpu.make_async_copy(ones_v, acc_sh.at[dst_v.at[NB - 1]], sems[1]).wait()
    plsc.subcore_barrier()
    pltpu.sync_copy(acc_sh.at[pl.ds(s * ROWS_PT, ROWS_PT)],
                    out_hbm.at[c, pl.ds(s * ROWS_PT, ROWS_PT)])


SCH = 16
NSC = NB // SCH


def _make_agg_kernel(D):
    @functools.partial(
        pl.kernel,
        out_type=jax.ShapeDtypeStruct((2, NPAD, D), jnp.float32),
        mesh=_mesh,
        scratch_types=[
            pltpu.VMEM((SCH, EPB), jnp.int32),
            pltpu.VMEM((SCH, EPB), jnp.int32),
            pltpu.VMEM((EPB, D), jnp.float32),
            pltpu.VMEM((EPB, D), jnp.float32),
            pltpu.VMEM_SHARED((NPAD, D), jnp.float32),
            pltpu.SemaphoreType.DMA,
            pltpu.SemaphoreType.DMA,
            pltpu.SemaphoreType.DMA,
            pltpu.SemaphoreType.DMA,
        ],
    )
    def agg(y_hbm, src_hbm, dst_hbm, zeros_hbm, out_hbm,
            src_v, dst_v, buf0, buf1, acc_sh, semg0, semg1, sems0, sems1):
        c = lax.axis_index("c")
        s = lax.axis_index("s")
        w = c * 16 + s
        pltpu.sync_copy(zeros_hbm.at[pl.ds(s * ROWS_PT, ROWS_PT)],
                        acc_sh.at[pl.ds(s * ROWS_PT, ROWS_PT)])
        plsc.subcore_barrier()

        bufs = (buf0, buf1)
        semg = (semg0, semg1)
        sems = (sems0, sems1)

        @pl.loop(0, NSC)
        def _(sc):
            pltpu.sync_copy(src_hbm.at[w, pl.ds(sc * SCH, SCH)], src_v)
            pltpu.sync_copy(dst_hbm.at[w, pl.ds(sc * SCH, SCH)], dst_v)
            pltpu.async_copy(y_hbm.at[src_v.at[0]], buf0, semg0)

            @pl.loop(0, SCH // 2)
            def _(g):
                for b in range(2):
                    j = g * 2 + b

                    @pl.when(j >= 1)
                    def _():
                        pltpu.make_async_copy(
                            bufs[1 - b], acc_sh.at[dst_v.at[j - 1]],
                            sems[1 - b]).wait()

                    @pl.when(j + 1 < SCH)
                    def _():
                        pltpu.async_copy(y_hbm.at[src_v.at[j + 1]],
                                         bufs[1 - b], semg[1 - b])

                    pltpu.make_async_copy(y_hbm.at[src_v.at[j]], bufs[b],
                                          semg[b]).wait()
                    pltpu.async_copy(bufs[b], acc_sh.at[dst_v.at[j]],
                                     sems[b], add=True)

            pltpu.make_async_copy(bufs[1], acc_sh.at[dst_v.at[SCH - 1]],
                                  sems[1]).wait()

        plsc.subcore_barrier()
        pltpu.sync_copy(acc_sh.at[pl.ds(s * ROWS_PT, ROWS_PT)],
                        out_hbm.at[c, pl.ds(s * ROWS_PT, ROWS_PT)])

    return agg


_agg128_kernel = _make_agg_kernel(128)


def _t0_body(degp_ref, x_ref, dinv_ref, y_ref):
    deg8 = degp_ref[0] + degp_ref[1] + 1.0
    dinv8 = lax.rsqrt(deg8)
    d = dinv8[:, 0:1]
    dinv_ref[...] = d
    y_ref[0:N, :] = x_ref[...] * d[0:N]
    y_ref[pl.ds(N, NPAD - N), :] = jnp.zeros((NPAD - N, 128), jnp.float32)


def _t1_body(sp_ref, y_ref, dinv_ref, w1_ref, b1_ref, w2_ref, z_ref, y2_ref):
    d = dinv_ref[...]
    agg = d * (sp_ref[0] + sp_ref[1]) + d * y_ref[...]
    h = jnp.dot(agg, w1_ref[...], preferred_element_type=jnp.float32) + b1_ref[...]
    h = jnp.where(h >= 0, h, 0.01 * h)
    z = jnp.dot(h, w2_ref[...], preferred_element_type=jnp.float32)
    z_ref[...] = z
    y2_ref[...] = z * d


def _t2_body(sp_ref, z_ref, dinv_ref, b2_ref, out_ref):
    d = dinv_ref[...]
    o = (d * (sp_ref[0][:, 0:4] + sp_ref[1][:, 0:4])
         + (d * d) * z_ref[:, 0:4] + b2_ref[...])
    m = jnp.max(o, axis=1, keepdims=True)
    e = jnp.exp(o - m)
    out_ref[...] = o - m - jnp.log(jnp.sum(e, axis=1, keepdims=True))


def kernel(in_feat, edge_index, W1, b1, W2, b2):
    src = edge_index[0].astype(jnp.int32)
    dst = edge_index[1].astype(jnp.int32)
    e = src.shape[0]
    pad_i = jnp.arange(EPAD - e, dtype=jnp.int32)
    pad_src = N + pad_i % (NPAD - N)
    pad_dst = pad_i % NPAD
    src_p = jnp.concatenate([src, pad_src]).reshape(NTILES, NB, EPB)
    dst_p = jnp.concatenate([dst, pad_dst]).reshape(NTILES, NB, EPB)

    ones1 = jnp.zeros((EPB, DEGW), jnp.float32).at[:, 0].set(1.0)
    zeros128 = jnp.zeros((NPAD, 128), jnp.float32)
    w2p = jnp.zeros((512, 128), jnp.float32).at[:, :4].set(W2)

    degp = _deg_kernel(dst_p, ones1, zeros128)

    dinv, y1 = pl.pallas_call(
        _t0_body,
        out_shape=[jax.ShapeDtypeStruct((NPAD, 1), jnp.float32),
                   jax.ShapeDtypeStruct((NPAD, 128), jnp.float32)],
    )(degp, in_feat)

    s1p = _agg128_kernel(y1, src_p, dst_p, zeros128)

    R = 1280
    z, y2 = pl.pallas_call(
        _t1_body,
        grid=(NPAD // R,),
        in_specs=[
            pl.BlockSpec((2, R, 128), lambda i: (0, i, 0)),
            pl.BlockSpec((R, 128), lambda i: (i, 0)),
            pl.BlockSpec((R, 1), lambda i: (i, 0)),
            pl.BlockSpec((128, 512), lambda i: (0, 0)),
            pl.BlockSpec((1, 512), lambda i: (0, 0)),
            pl.BlockSpec((512, 128), lambda i: (0, 0)),
        ],
        out_specs=[
            pl.BlockSpec((R, 128), lambda i: (i, 0)),
            pl.BlockSpec((R, 128), lambda i: (i, 0)),
        ],
        out_shape=[jax.ShapeDtypeStruct((NPAD, 128), jnp.float32),
                   jax.ShapeDtypeStruct((NPAD, 128), jnp.float32)],
    )(s1p, y1, dinv, W1, b1.reshape(1, 512), w2p)

    s2p = _agg128_kernel(y2, src_p, dst_p, zeros128)

    out = pl.pallas_call(
        _t2_body,
        out_shape=jax.ShapeDtypeStruct((NPAD, 4), jnp.float32),
    )(s2p, z, dinv, b2.reshape(1, 4))
    return out[:N]
